# baseline (speedup 1.0000x reference)
import functools
import math

import jax
import jax.numpy as jnp
from jax import lax
from jax.experimental import pallas as pl
from jax.experimental.pallas import tpu as pltpu
from jax.experimental.pallas import tpu_sc as plsc

D_MODEL = 64
SCALE = 8.0
LANES = 16
CT = 128
NB = 4


def _make_sc_gather(BATCH, HIST, V):
    info = plsc.get_sparse_core_info()
    NC, NS = info.num_cores, info.num_subcores
    NW = NC * NS
    assert BATCH % (CT * NW) == 0
    nch = HIST
    tg = CT // LANES

    mesh = plsc.VectorSubcoreMesh(core_axis_name="c", subcore_axis_name="s")

    @functools.partial(
        pl.kernel,
        out_type=jax.ShapeDtypeStruct((HIST, D_MODEL, BATCH), jnp.float32),
        mesh=mesh,
        scratch_types=[
            pltpu.VMEM((nch, CT), jnp.int32),
            [pltpu.VMEM((CT,), jnp.int32) for _ in range(NB)],
            [pltpu.VMEM((CT,), jnp.int32) for _ in range(NB)],
            [pltpu.VMEM((CT, 2 * D_MODEL), jnp.float32) for _ in range(NB)],
            [pltpu.VMEM((D_MODEL, CT), jnp.float32) for _ in range(NB)],
            [pltpu.SemaphoreType.DMA for _ in range(NB)],
            [pltpu.SemaphoreType.DMA for _ in range(NB)],
        ],
        compiler_params=pltpu.CompilerParams(needs_layout_passes=False),
    )
    def body(tab_hbm, idx_hbm, out_hbm, idx_all, pvs, hvs, bufs, tbufs,
             gsems, ssems):
        wid = lax.axis_index("s") * NC + lax.axis_index("c")
        bcol = wid * CT

        for th in range(nch // 8):
            pltpu.sync_copy(
                idx_hbm.at[pl.ds(th * 8, 8), pl.ds(bcol, CT)],
                idx_all.at[pl.ds(th * 8, 8)],
            )

        def gather_desc(s):
            return pltpu.make_async_copy(tab_hbm.at[pvs[s]], bufs[s], gsems[s])

        def scatter_descs(h, s):
            return [
                pltpu.make_async_copy(
                    tbufs[s].at[pl.ds(tc * 8, 8)],
                    out_hbm.at[h, pl.ds(tc * 8, 8), pl.ds(bcol, CT)],
                    ssems[s],
                )
                for tc in range(D_MODEL // 8)
            ]

        def pre(h, s):
            @pl.when(h >= NB)
            def _():
                for d in scatter_descs(h - NB, s):
                    d.wait()

            for g in range(tg):
                sl = pl.ds(g * LANES, LANES)
                v = idx_all[h, sl]
                pvs[s][sl] = v >> 1
                hvs[s][sl] = (v & 1) << 6
            gather_desc(s).start()

        def post(h, s):
            gather_desc(s).wait()
            buf, tbuf, hv_ref = bufs[s], tbufs[s], hvs[s]
            ci = lax.iota(jnp.int32, LANES)
            rowvs = [lax.iota(jnp.int32, LANES) + (g * LANES) for g in range(tg)]
            hvv = [hv_ref[pl.ds(g * LANES, LANES)] for g in range(tg)]

            @plsc.parallel_loop(0, LANES, unroll=1)
            def _sel(k):
                w = (ci + jnp.full((LANES,), k, jnp.int32)) & (LANES - 1)
                for j in range(D_MODEL // LANES):
                    colv = w + (j * LANES)
                    for g in range(tg):
                        v = plsc.load_gather(buf, [rowvs[g], hvv[g] + colv])
                        plsc.store_scatter(tbuf, [colv, rowvs[g]], v * SCALE)

            for d in scatter_descs(h, s):
                d.start()

        def step(h, s_pre, s_post):
            @pl.when(h < nch)
            def _():
                pre(h, s_pre)

            h2 = h - 2

            @pl.when(jnp.logical_and(h2 >= 0, h2 < nch))
            def _():
                post(h2, s_post)

        n_steps = nch + 2
        n_rounds = (n_steps + NB - 1) // NB

        def round_body(k, carry):
            for t in range(NB):
                step(k * NB + t, t, (t + 2) % NB)
            return carry

        lax.fori_loop(0, n_rounds, round_body, 0)

        for h in range(nch - NB, nch):
            for d in scatter_descs(h, h % NB):
                d.wait()

    return body


def kernel(token_ids, embedding_weights):
    BATCH, HIST = token_ids.shape
    V = embedding_weights.shape[0]
    tab2 = jnp.concatenate(
        [embedding_weights[0::2], embedding_weights[1::2]], axis=1)
    idxT = token_ids.T
    out3 = _make_sc_gather(BATCH, HIST, V)(tab2, idxT)
    return out3.transpose(2, 0, 1)


# speedup vs baseline: 10.4031x; 10.4031x over previous
import functools
import math

import jax
import jax.numpy as jnp
from jax import lax
from jax.experimental import pallas as pl
from jax.experimental.pallas import tpu as pltpu
from jax.experimental.pallas import tpu_sc as plsc

D_MODEL = 64
SCALE = 8.0
LANES = 16
CT = 128
NB = 4


def _make_sc_gather(BATCH, HIST, V):
    info = plsc.get_sparse_core_info()
    NC, NS = info.num_cores, info.num_subcores
    NW = NC * NS
    assert BATCH % (CT * NW) == 0
    nch = HIST
    tg = CT // LANES

    mesh = plsc.VectorSubcoreMesh(core_axis_name="c", subcore_axis_name="s")

    @functools.partial(
        pl.kernel,
        out_type=jax.ShapeDtypeStruct((HIST, D_MODEL, BATCH), jnp.float32),
        mesh=mesh,
        scratch_types=[
            pltpu.VMEM((nch, CT), jnp.int32),
            [pltpu.VMEM((CT,), jnp.int32) for _ in range(NB)],
            [pltpu.VMEM((CT,), jnp.int32) for _ in range(NB)],
            [pltpu.VMEM((CT, 2 * D_MODEL), jnp.float32) for _ in range(NB)],
            [pltpu.VMEM((D_MODEL, CT), jnp.float32) for _ in range(NB)],
            [pltpu.SemaphoreType.DMA for _ in range(NB)],
            [pltpu.SemaphoreType.DMA for _ in range(NB)],
        ],
        compiler_params=pltpu.CompilerParams(needs_layout_passes=False),
    )
    def body(tab_hbm, idx_hbm, out_hbm, idx_all, pvs, hvs, bufs, tbufs,
             gsems, ssems):
        wid = lax.axis_index("s") * NC + lax.axis_index("c")
        bcol = wid * CT

        for th in range(nch // 8):
            pltpu.sync_copy(
                idx_hbm.at[pl.ds(th * 8, 8), pl.ds(bcol, CT)],
                idx_all.at[pl.ds(th * 8, 8)],
            )

        def gather_desc(s):
            return pltpu.make_async_copy(tab_hbm.at[pvs[s]], bufs[s], gsems[s])

        def scatter_descs(h, s):
            return [
                pltpu.make_async_copy(
                    tbufs[s].at[pl.ds(tc * 8, 8)],
                    out_hbm.at[h, pl.ds(tc * 8, 8), pl.ds(bcol, CT)],
                    ssems[s],
                )
                for tc in range(D_MODEL // 8)
            ]

        def pre(h, s):
            @pl.when(h >= NB)
            def _():
                for d in scatter_descs(h - NB, s):
                    d.wait()

            for g in range(tg):
                sl = pl.ds(g * LANES, LANES)
                v = idx_all[h, sl]
                pvs[s][sl] = v >> 1
                hvs[s][sl] = (v & 1) << 6
            gather_desc(s).start()

        def post(h, s):
            gather_desc(s).wait()
            buf, tbuf, hv_ref = bufs[s], tbufs[s], hvs[s]
            ci = lax.iota(jnp.int32, LANES)
            rowvs = [lax.iota(jnp.int32, LANES) + (g * LANES) for g in range(tg)]
            hvv = [hv_ref[pl.ds(g * LANES, LANES)] for g in range(tg)]

            @plsc.parallel_loop(0, LANES, unroll=1)
            def _sel(k):
                w = (ci + jnp.full((LANES,), k, jnp.int32)) & (LANES - 1)
                for j in range(D_MODEL // LANES):
                    colv = w + (j * LANES)
                    for g in range(tg):
                        v = plsc.load_gather(buf, [rowvs[g], hvv[g] + colv])
                        plsc.store_scatter(tbuf, [colv, rowvs[g]], v * SCALE)

            for d in scatter_descs(h, s):
                d.start()

        def step(h, s_pre, s_post):
            @pl.when(h < nch)
            def _():
                pre(h, s_pre)

            h2 = h - 2

            @pl.when(jnp.logical_and(h2 >= 0, h2 < nch))
            def _():
                post(h2, s_post)

        n_steps = nch + 2
        n_rounds = (n_steps + NB - 1) // NB

        def round_body(k, carry):
            for t in range(NB):
                step(k * NB + t, t, (t + 2) % NB)
            return carry

        lax.fori_loop(0, n_rounds, round_body, 0)

        for h in range(nch - NB, nch):
            for d in scatter_descs(h, h % NB):
                d.wait()

    return body


def kernel(token_ids, embedding_weights):
    BATCH, HIST = token_ids.shape
    V = embedding_weights.shape[0]
    tab2 = embedding_weights.reshape(V // 2, 2 * D_MODEL)
    idxT = token_ids.T
    out3 = _make_sc_gather(BATCH, HIST, V)(tab2, idxT)
    return out3.transpose(2, 0, 1)


# speedup vs baseline: 10.4826x; 1.0076x over previous
import functools
import math

import jax
import jax.numpy as jnp
from jax import lax
from jax.experimental import pallas as pl
from jax.experimental.pallas import tpu as pltpu
from jax.experimental.pallas import tpu_sc as plsc

D_MODEL = 64
SCALE = 8.0
LANES = 16
CT = 128
NB = 4


def _make_sc_gather(BATCH, HIST, V):
    info = plsc.get_sparse_core_info()
    NC, NS = info.num_cores, info.num_subcores
    NW = NC * NS
    assert BATCH % (CT * NW) == 0
    nch = HIST
    tg = CT // LANES

    mesh = plsc.VectorSubcoreMesh(core_axis_name="c", subcore_axis_name="s")

    @functools.partial(
        pl.kernel,
        out_type=jax.ShapeDtypeStruct((HIST, D_MODEL, BATCH), jnp.float32),
        mesh=mesh,
        scratch_types=[
            pltpu.VMEM((nch, CT), jnp.int32),
            [pltpu.VMEM((CT,), jnp.int32) for _ in range(NB)],
            [pltpu.VMEM((CT,), jnp.int32) for _ in range(NB)],
            [pltpu.VMEM((CT, 2 * D_MODEL), jnp.float32) for _ in range(NB)],
            [pltpu.VMEM((D_MODEL, CT), jnp.float32) for _ in range(NB)],
            [pltpu.SemaphoreType.DMA for _ in range(NB)],
            [pltpu.SemaphoreType.DMA for _ in range(NB)],
        ],
        compiler_params=pltpu.CompilerParams(needs_layout_passes=False),
    )
    def body(tab_hbm, idx_hbm, out_hbm, idx_all, pvs, hvs, bufs, tbufs,
             gsems, ssems):
        wid = lax.axis_index("s") * NC + lax.axis_index("c")
        bcol = wid * CT

        for th in range(nch // 8):
            pltpu.sync_copy(
                idx_hbm.at[pl.ds(th * 8, 8), pl.ds(bcol, CT)],
                idx_all.at[pl.ds(th * 8, 8)],
            )

        def gather_desc(s):
            return pltpu.make_async_copy(tab_hbm.at[pvs[s]], bufs[s], gsems[s])

        def scatter_descs(h, s):
            return [
                pltpu.make_async_copy(
                    tbufs[s],
                    out_hbm.at[h, :, pl.ds(bcol, CT)],
                    ssems[s],
                )
            ]

        def pre(h, s):
            @pl.when(h >= NB)
            def _():
                for d in scatter_descs(h - NB, s):
                    d.wait()

            for g in range(tg):
                sl = pl.ds(g * LANES, LANES)
                v = idx_all[h, sl]
                pvs[s][sl] = v >> 1
                hvs[s][sl] = (v & 1) << 6
            gather_desc(s).start()

        def post(h, s):
            gather_desc(s).wait()
            buf, tbuf, hv_ref = bufs[s], tbufs[s], hvs[s]
            ci = lax.iota(jnp.int32, LANES)
            rowvs = [lax.iota(jnp.int32, LANES) + (g * LANES) for g in range(tg)]
            hvv = [hv_ref[pl.ds(g * LANES, LANES)] for g in range(tg)]

            @plsc.parallel_loop(0, LANES, unroll=1)
            def _sel(k):
                w = (ci + jnp.full((LANES,), k, jnp.int32)) & (LANES - 1)
                for j in range(D_MODEL // LANES):
                    colv = w + (j * LANES)
                    for g in range(tg):
                        v = plsc.load_gather(buf, [rowvs[g], hvv[g] + colv])
                        plsc.store_scatter(tbuf, [colv, rowvs[g]], v * SCALE)

            for d in scatter_descs(h, s):
                d.start()

        def step(h, s_pre, s_post):
            @pl.when(h < nch)
            def _():
                pre(h, s_pre)

            h2 = h - 2

            @pl.when(jnp.logical_and(h2 >= 0, h2 < nch))
            def _():
                post(h2, s_post)

        n_steps = nch + 2
        n_rounds = (n_steps + NB - 1) // NB

        def round_body(k, carry):
            for t in range(NB):
                step(k * NB + t, t, (t + 2) % NB)
            return carry

        lax.fori_loop(0, n_rounds, round_body, 0)

        for h in range(nch - NB, nch):
            for d in scatter_descs(h, h % NB):
                d.wait()

    return body


def kernel(token_ids, embedding_weights):
    BATCH, HIST = token_ids.shape
    V = embedding_weights.shape[0]
    tab2 = embedding_weights.reshape(V // 2, 2 * D_MODEL)
    idxT = token_ids.T
    out3 = _make_sc_gather(BATCH, HIST, V)(tab2, idxT)
    return out3.transpose(2, 0, 1)


# speedup vs baseline: 10.6528x; 1.0162x over previous
import functools
import math

import jax
import jax.numpy as jnp
from jax import lax
from jax.experimental import pallas as pl
from jax.experimental.pallas import tpu as pltpu
from jax.experimental.pallas import tpu_sc as plsc

D_MODEL = 64
SCALE = 8.0
LANES = 16
CT = 128
NB = 4


def _make_sc_gather(BATCH, HIST, V):
    info = plsc.get_sparse_core_info()
    NC, NS = info.num_cores, info.num_subcores
    NW = NC * NS
    assert BATCH % (CT * NW) == 0
    nch = HIST
    tg = CT // LANES

    mesh = plsc.VectorSubcoreMesh(core_axis_name="c", subcore_axis_name="s")

    @functools.partial(
        pl.kernel,
        out_type=jax.ShapeDtypeStruct((HIST, D_MODEL, BATCH), jnp.float32),
        mesh=mesh,
        scratch_types=[
            pltpu.VMEM((nch, CT), jnp.int32),
            [pltpu.VMEM((CT,), jnp.int32) for _ in range(NB)],
            [pltpu.VMEM((CT,), jnp.int32) for _ in range(NB)],
            [pltpu.VMEM((CT, 2 * D_MODEL), jnp.float32) for _ in range(NB)],
            [pltpu.VMEM((D_MODEL, CT), jnp.float32) for _ in range(NB)],
            [pltpu.SemaphoreType.DMA for _ in range(NB)],
            [pltpu.SemaphoreType.DMA for _ in range(NB)],
        ],
        compiler_params=pltpu.CompilerParams(needs_layout_passes=False),
    )
    def body(tab_hbm, idx_hbm, out_hbm, idx_all, pvs, hvs, bufs, tbufs,
             gsems, ssems):
        wid = lax.axis_index("s") * NC + lax.axis_index("c")
        bcol = wid * CT

        pltpu.sync_copy(idx_hbm.at[:, pl.ds(bcol, CT)], idx_all)

        def gather_desc(s):
            return pltpu.make_async_copy(tab_hbm.at[pvs[s]], bufs[s], gsems[s])

        def scatter_descs(h, s):
            return [
                pltpu.make_async_copy(
                    tbufs[s],
                    out_hbm.at[h, :, pl.ds(bcol, CT)],
                    ssems[s],
                )
            ]

        def pre(h, s):
            @pl.when(h >= NB)
            def _():
                for d in scatter_descs(h - NB, s):
                    d.wait()

            for g in range(tg):
                sl = pl.ds(g * LANES, LANES)
                v = idx_all[h, sl]
                pvs[s][sl] = v >> 1
                hvs[s][sl] = (v & 1) << 6
            gather_desc(s).start()

        def post(h, s):
            gather_desc(s).wait()
            buf, tbuf, hv_ref = bufs[s], tbufs[s], hvs[s]
            ci = lax.iota(jnp.int32, LANES)
            rowvs = [lax.iota(jnp.int32, LANES) + (g * LANES) for g in range(tg)]
            hvv = [hv_ref[pl.ds(g * LANES, LANES)] for g in range(tg)]

            @plsc.parallel_loop(0, LANES, unroll=1)
            def _sel(k):
                w = (ci + jnp.full((LANES,), k, jnp.int32)) & (LANES - 1)
                for j in range(D_MODEL // LANES):
                    colv = w + (j * LANES)
                    for g in range(tg):
                        v = plsc.load_gather(buf, [rowvs[g], hvv[g] + colv])
                        plsc.store_scatter(tbuf, [colv, rowvs[g]], v * SCALE)

            for d in scatter_descs(h, s):
                d.start()

        def step(h, s_pre, s_post):
            @pl.when(h < nch)
            def _():
                pre(h, s_pre)

            h2 = h - 2

            @pl.when(jnp.logical_and(h2 >= 0, h2 < nch))
            def _():
                post(h2, s_post)

        n_steps = nch + 2
        n_rounds = (n_steps + NB - 1) // NB

        def round_body(k, carry):
            for t in range(NB):
                step(k * NB + t, t, (t + 2) % NB)
            return carry

        lax.fori_loop(0, n_rounds, round_body, 0)

        for h in range(nch - NB, nch):
            for d in scatter_descs(h, h % NB):
                d.wait()

    return body


def kernel(token_ids, embedding_weights):
    BATCH, HIST = token_ids.shape
    V = embedding_weights.shape[0]
    tab2 = embedding_weights.reshape(V // 2, 2 * D_MODEL)
    idxT = token_ids.T
    out3 = _make_sc_gather(BATCH, HIST, V)(tab2, idxT)
    return out3.transpose(2, 0, 1)
